# baseline (speedup 1.0000x reference)
import functools
import jax
import jax.numpy as jnp
from jax import lax
from jax.experimental import pallas as pl
from jax.experimental.pallas import tpu as pltpu
from jax.experimental.pallas import tpu_sc as plsc

_N = 10000
_NPAD = 10240
_E = 320000
_D = 128
_G = 64
_H = 128
_RATIO = 0.8
_NW = 32
_EW = _E // _NW
_TPW = _NPAD // _NW

_mesh = functools.partial(plsc.VectorSubcoreMesh,
                          core_axis_name="c", subcore_axis_name="s")
_cparams = functools.partial(pltpu.CompilerParams, needs_layout_passes=False)


def _wid():
    return lax.axis_index("c") * 16 + lax.axis_index("s")


def _splat_i32(x):
    return jnp.full((16,), x, jnp.int32)


def _make_scalar_pass(n_tables, val_fn, edge_out):
    CH = 2000
    NCH = _EW // CH

    def body(*refs):
        row_hbm, col_hbm, zeros_hbm = refs[0], refs[1], refs[2]
        tab_hbm = refs[3:3 + n_tables]
        i = 3 + n_tables
        out_hbm = refs[i]
        oute_hbm = refs[i + 1] if edge_out else None
        i += 2 if edge_out else 1
        rowbuf, colbuf, valbuf = refs[i], refs[i + 1], refs[i + 2]
        tabs = refs[i + 3:i + 3 + n_tables]
        acc = refs[i + 3 + n_tables]

        cid = lax.axis_index("c")
        sid = lax.axis_index("s")
        stripe = _NPAD // 16
        for t_hbm, t_v in zip(tab_hbm, tabs):
            pltpu.sync_copy(t_hbm, t_v)
        pltpu.sync_copy(zeros_hbm.at[pl.ds(sid * stripe, stripe)],
                        acc.at[pl.ds(sid * stripe, stripe)])
        plsc.subcore_barrier()

        base = _wid() * _EW

        def chunk(ci, c):
            pltpu.sync_copy(row_hbm.at[pl.ds(base + ci * CH, CH)], rowbuf)
            pltpu.sync_copy(col_hbm.at[pl.ds(base + ci * CH, CH)], colbuf)

            def grp(j, c2):
                r16 = rowbuf[pl.ds(j * 16, 16)]
                c16 = colbuf[pl.ds(j * 16, 16)]
                valbuf[pl.ds(j * 16, 16)] = val_fn(r16, c16, tabs)
                return c2
            lax.fori_loop(0, CH // 16, grp, 0)
            pltpu.sync_copy(valbuf, acc.at[colbuf], add=True)
            if edge_out:
                pltpu.sync_copy(valbuf,
                                oute_hbm.at[pl.ds(base + ci * CH, CH)])
            return c
        lax.fori_loop(0, NCH, chunk, 0)
        plsc.subcore_barrier()
        pltpu.sync_copy(acc.at[pl.ds(sid * stripe, stripe)],
                        out_hbm.at[cid, pl.ds(sid * stripe, stripe)])

    scratch = ([pltpu.VMEM((CH,), jnp.int32), pltpu.VMEM((CH,), jnp.int32),
                pltpu.VMEM((CH,), jnp.float32)]
               + [pltpu.VMEM((_NPAD,), jnp.float32) for _ in range(n_tables)]
               + [pltpu.VMEM_SHARED((_NPAD,), jnp.float32)])
    out_type = jax.ShapeDtypeStruct((2, _NPAD), jnp.float32)
    if edge_out:
        out_type = (out_type, jax.ShapeDtypeStruct((_E,), jnp.float32))
    return pl.kernel(
        body,
        out_type=out_type,
        mesh=_mesh(),
        scratch_types=scratch,
        compiler_params=_cparams(),
    )


def _deg_val(r16, c16, tabs):
    del r16, c16, tabs
    return jnp.ones((16,), jnp.float32)


def _escore_val(r16, c16, tabs):
    t1, pj, m = tabs
    s = plsc.load_gather(t1, [c16]) + plsc.load_gather(pj, [r16])
    s = jnp.where(s > 0, s, jnp.float32(0.2) * s)
    return jnp.exp(s - plsc.load_gather(m, [c16]))


def _agg_val(r16, c16, tabs):
    del c16
    return plsc.load_gather(tabs[0], [r16])


_CHS = 10000
_NR = _E // _CHS


def _make_dstadd(weighted):
    _GB = 128 if weighted else 256

    def body(*refs):
        if weighted:
            z_hbm, row_hbm, col_hbm, w_hbm, out_hbm = refs[:5]
            rowb, colb, wb, crow, clc, cw, rows, acc, sem = refs[5:]
        else:
            z_hbm, row_hbm, col_hbm, out_hbm = refs[:4]
            rowb, colb, crow, clc, rows, acc, sem = refs[4:]
            wb = cw = None

        zero16f = jnp.zeros((16,), jnp.float32)
        zero16i = jnp.zeros((16,), jnp.int32)

        def iacc(i, c):
            acc[pl.ds(i * 16, 16)] = zero16f
            return c
        lax.fori_loop(0, _TPW * _D // 16, iacc, 0)

        def icrow(i, c):
            crow[pl.ds(i * 16, 16)] = zero16i
            return c
        lax.fori_loop(0, _CHS // 16, icrow, 0)

        lo = _wid() * _TPW
        hi = lo + _TPW

        def rnd(ri, c0):
            pltpu.sync_copy(row_hbm.at[pl.ds(ri * _CHS, _CHS)], rowb)
            pltpu.sync_copy(col_hbm.at[pl.ds(ri * _CHS, _CHS)], colb)
            if weighted:
                pltpu.sync_copy(w_hbm.at[pl.ds(ri * _CHS, _CHS)], wb)

            def grp(j, cnt):
                c16 = colb[pl.ds(j * 16, 16)]
                r16 = rowb[pl.ds(j * 16, 16)]
                msk = (c16 >= lo) & (c16 < hi)
                pos = (plsc.cumsum(msk.astype(jnp.int32))
                       + _splat_i32(cnt - 1))
                plsc.store_scatter(crow, [pos], r16, mask=msk)
                plsc.store_scatter(clc, [pos], c16 - lo, mask=msk)
                if weighted:
                    plsc.store_scatter(cw, [pos], wb[pl.ds(j * 16, 16)],
                                       mask=msk)
                return cnt + jnp.max(plsc.all_reduce_population_count(msk))
            cnt = lax.fori_loop(0, _CHS // 16, grp, 0)

            def batch(b, c2):
                s = b * _GB
                pltpu.async_copy(z_hbm.at[crow.at[pl.ds(s, _GB)]],
                                 rows, sem).wait()
                size = jnp.minimum(_GB, cnt - s)

                def edge(e, c3):
                    se16 = _splat_i32(s + e)
                    lc = plsc.load_gather(clc, [se16])
                    e16 = _splat_i32(e)
                    wv = plsc.load_gather(cw, [se16]) if weighted else None
                    for f in range(_D // 16):
                        fidx = f * 16 + lax.iota(jnp.int32, 16)
                        cur = plsc.load_gather(acc, [lc * _D + fidx])
                        rv = plsc.load_gather(rows, [e16, fidx])
                        if weighted:
                            rv = rv * wv
                        plsc.store_scatter(acc, [lc * _D + fidx], cur + rv)
                    return c3
                lax.fori_loop(0, size, edge, 0)
                return c2
            lax.fori_loop(0, (cnt + _GB - 1) // _GB, batch, 0)
            return c0
        lax.fori_loop(0, _NR, rnd, 0)
        pltpu.sync_copy(acc, out_hbm.at[pl.ds(lo * _D, _TPW * _D)])

    scratch = [pltpu.VMEM((_CHS,), jnp.int32), pltpu.VMEM((_CHS,), jnp.int32)]
    if weighted:
        scratch.append(pltpu.VMEM((_CHS,), jnp.float32))
    scratch += [pltpu.VMEM((_CHS,), jnp.int32), pltpu.VMEM((_CHS,), jnp.int32)]
    if weighted:
        scratch.append(pltpu.VMEM((_CHS,), jnp.float32))
    scratch += [pltpu.VMEM((_GB, _D), jnp.float32),
                pltpu.VMEM((_TPW * _D,), jnp.float32),
                pltpu.SemaphoreType.DMA]
    return pl.kernel(
        body,
        out_type=jax.ShapeDtypeStruct((_NPAD * _D,), jnp.float32),
        mesh=_mesh(),
        scratch_types=scratch,
        compiler_params=_cparams(),
    )


def _make_maxpass():
    _GB = 256

    def body(xp_hbm, row_hbm, col_hbm, pj_hbm, outx_hbm, outp_hbm,
             rowb, colb, crow, clc, rows, pjtab, accx, accp, sem):
        pltpu.sync_copy(pj_hbm, pjtab)
        neg = jnp.full((16,), -jnp.inf, jnp.float32)
        zero16i = jnp.zeros((16,), jnp.int32)

        def iacc(i, c):
            accx[pl.ds(i * 16, 16)] = neg
            return c
        lax.fori_loop(0, _TPW * _D // 16, iacc, 0)

        def iaccp(i, c):
            accp[pl.ds(i * 16, 16)] = neg
            return c
        lax.fori_loop(0, _TPW // 16, iaccp, 0)

        def icrow(i, c):
            crow[pl.ds(i * 16, 16)] = zero16i
            return c
        lax.fori_loop(0, _CHS // 16, icrow, 0)

        lo = _wid() * _TPW
        hi = lo + _TPW

        def rnd(ri, c):
            pltpu.sync_copy(row_hbm.at[pl.ds(ri * _CHS, _CHS)], rowb)
            pltpu.sync_copy(col_hbm.at[pl.ds(ri * _CHS, _CHS)], colb)

            def grp(j, cnt):
                c16 = colb[pl.ds(j * 16, 16)]
                r16 = rowb[pl.ds(j * 16, 16)]
                msk = (c16 >= lo) & (c16 < hi)
                pos = (plsc.cumsum(msk.astype(jnp.int32))
                       + _splat_i32(cnt - 1))
                plsc.store_scatter(crow, [pos], r16, mask=msk)
                plsc.store_scatter(clc, [pos], c16 - lo, mask=msk)
                return cnt + jnp.max(plsc.all_reduce_population_count(msk))
            cnt = lax.fori_loop(0, _CHS // 16, grp, 0)

            def batch(b, c2):
                s = b * _GB
                pltpu.async_copy(xp_hbm.at[crow.at[pl.ds(s, _GB)]],
                                 rows, sem).wait()
                size = jnp.minimum(_GB, cnt - s)

                def edge(e, c3):
                    se16 = _splat_i32(s + e)
                    lc = plsc.load_gather(clc, [se16])
                    crow16 = plsc.load_gather(crow, [se16])
                    e16 = _splat_i32(e)
                    for f in range(_D // 16):
                        fidx = f * 16 + lax.iota(jnp.int32, 16)
                        cur = plsc.load_gather(accx, [lc * _D + fidx])
                        rv = plsc.load_gather(rows, [e16, fidx])
                        plsc.store_scatter(accx, [lc * _D + fidx],
                                           jnp.maximum(cur, rv))
                    pv = plsc.load_gather(pjtab, [crow16])
                    cp = plsc.load_gather(accp, [lc])
                    plsc.store_scatter(accp, [lc], jnp.maximum(cp, pv))
                    return c3
                lax.fori_loop(0, size, edge, 0)
                return c2
            lax.fori_loop(0, (cnt + _GB - 1) // _GB, batch, 0)
            return c
        lax.fori_loop(0, _NR, rnd, 0)

        pltpu.sync_copy(accx, outx_hbm.at[pl.ds(lo * _D, _TPW * _D)])
        pltpu.sync_copy(accp, outp_hbm.at[pl.ds(lo, _TPW)])

    return pl.kernel(
        body,
        out_type=(jax.ShapeDtypeStruct((_NPAD * _D,), jnp.float32),
                  jax.ShapeDtypeStruct((_NPAD,), jnp.float32)),
        mesh=_mesh(),
        scratch_types=[
            pltpu.VMEM((_CHS,), jnp.int32), pltpu.VMEM((_CHS,), jnp.int32),
            pltpu.VMEM((_CHS,), jnp.int32), pltpu.VMEM((_CHS,), jnp.int32),
            pltpu.VMEM((_GB, _D), jnp.float32),
            pltpu.VMEM((_NPAD,), jnp.float32),
            pltpu.VMEM((_TPW * _D,), jnp.float32),
            pltpu.VMEM((_TPW,), jnp.float32),
            pltpu.SemaphoreType.DMA,
        ],
        compiler_params=_cparams(),
    )


_MMB = 2048


def _mm_kernel(x_ref, w_ref, o_ref):
    o_ref[...] = jnp.dot(x_ref[...], w_ref[...],
                         preferred_element_type=jnp.float32)


def _tc_mm(x, w):
    M = x.shape[0]
    blk = _MMB if M % _MMB == 0 else M
    return pl.pallas_call(
        _mm_kernel,
        grid=(M // blk,),
        in_specs=[pl.BlockSpec((blk, x.shape[1]), lambda i: (i, 0)),
                  pl.BlockSpec(w.shape, lambda i: (0, 0))],
        out_specs=pl.BlockSpec((blk, w.shape[1]), lambda i: (i, 0)),
        out_shape=jax.ShapeDtypeStruct((M, w.shape[1]), jnp.float32),
    )(x, w)


def _sum32_kernel(p_ref, o_ref):
    o_ref[...] = jnp.sum(p_ref[...], axis=0, keepdims=True)


def _tc_sum32(p):
    return pl.pallas_call(
        _sum32_kernel,
        out_shape=jax.ShapeDtypeStruct((1, _NPAD), jnp.float32),
    )(p)[0]


_RBI = 1024
_RBJ = 512


def _rank_kernel(kc_ref, bc_ref, ic_ref, kr_ref, br_ref, ir_ref, o_ref):
    ki, bi, ii = kc_ref[...], bc_ref[...], ic_ref[...]
    kl = jnp.zeros((_RBI, 1), jnp.float32)
    sb = jnp.zeros((_RBI, 1), jnp.float32)
    cb = jnp.zeros((_RBI, 1), jnp.float32)

    def jstep(j, carry):
        kl, sb, cb = carry
        kj = kr_ref[:, pl.ds(j * _RBJ, _RBJ)]
        bj = br_ref[:, pl.ds(j * _RBJ, _RBJ)]
        ij = ir_ref[:, pl.ds(j * _RBJ, _RBJ)]
        less = (kj < ki) | ((kj == ki) & (ij < ii))
        kl = kl + jnp.sum(less.astype(jnp.float32), axis=1, keepdims=True)
        sb = sb + jnp.sum((bj < bi).astype(jnp.float32), axis=1,
                          keepdims=True)
        cb = cb + jnp.sum((bj == bi).astype(jnp.float32), axis=1,
                          keepdims=True)
        return kl, sb, cb
    kl, sb, cb = lax.fori_loop(0, _NPAD // _RBJ, jstep, (kl, sb, cb))
    kper = jnp.ceil(jnp.float32(_RATIO) * cb)
    o_ref[...] = ((kl - sb) < kper).astype(jnp.float32)


def _tc_rank(key, batchf, idxf):
    kc = key[:, None]
    bc = batchf[:, None]
    ic = idxf[:, None]
    kr = key[None, :]
    br = batchf[None, :]
    ir = idxf[None, :]
    return pl.pallas_call(
        _rank_kernel,
        grid=(_NPAD // _RBI,),
        in_specs=[pl.BlockSpec((_RBI, 1), lambda i: (i, 0)),
                  pl.BlockSpec((_RBI, 1), lambda i: (i, 0)),
                  pl.BlockSpec((_RBI, 1), lambda i: (i, 0)),
                  pl.BlockSpec((1, _NPAD), lambda i: (0, 0)),
                  pl.BlockSpec((1, _NPAD), lambda i: (0, 0)),
                  pl.BlockSpec((1, _NPAD), lambda i: (0, 0))],
        out_specs=pl.BlockSpec((_RBI, 1), lambda i: (i, 0)),
        out_shape=jax.ShapeDtypeStruct((_NPAD, 1), jnp.float32),
    )(kc, bc, ic, kr, br, ir)[:, 0]


def _readout_kernel(x_ref, keep_ref, b_ref, o_ref):
    g0 = pl.program_id(0) * 8
    x = x_ref[...]
    keepm = keep_ref[...] > 0.5
    b = b_ref[...]
    rows = []
    for k in range(8):
        gval = (g0 + k).astype(jnp.float32)
        mask = (b == gval) & keepm
        kcnt = jnp.sum(mask.astype(jnp.float32))
        xm = jnp.sum(jnp.where(mask, x, 0.0), axis=0)
        xmx = jnp.max(jnp.where(mask, x, -jnp.inf), axis=0)
        rows.append(jnp.concatenate([xm / jnp.maximum(kcnt, 1.0), xmx],
                                    axis=-1))
    o_ref[...] = jnp.stack(rows)


def _tc_readout(x_out, keep, batchf):
    return pl.pallas_call(
        _readout_kernel,
        grid=(_G // 8,),
        in_specs=[pl.BlockSpec((_NPAD, _D), lambda g: (0, 0)),
                  pl.BlockSpec((_NPAD, 1), lambda g: (0, 0)),
                  pl.BlockSpec((_NPAD, 1), lambda g: (0, 0))],
        out_specs=pl.BlockSpec((8, 2 * _D), lambda g: (g, 0)),
        out_shape=jax.ShapeDtypeStruct((_G, 2 * _D), jnp.float32),
    )(x_out, keep[:, None], batchf[:, None])


def _head_kernel(xs_ref, wl1_ref, bl1_ref, wl2_ref, bl2_ref, out_ref):
    xs = xs_ref[...]
    z = jnp.maximum(jnp.dot(xs, wl1_ref[...],
                            preferred_element_type=jnp.float32)
                    + bl1_ref[...], 0.0)
    z = jnp.dot(z, wl2_ref[...], preferred_element_type=jnp.float32) \
        + bl2_ref[...]
    m = jnp.max(z, axis=-1, keepdims=True)
    e = jnp.exp(z - m)
    lse = jnp.log(jnp.sum(e, axis=-1, keepdims=True)) + m
    out_ref[...] = z - lse


def _head(xs, Wl1, bl1, Wl2, bl2):
    return pl.pallas_call(
        _head_kernel,
        out_shape=jax.ShapeDtypeStruct((_G, Wl2.shape[1]), jnp.float32),
    )(xs, Wl1, bl1[None, :], Wl2, bl2[None, :])


_sc_deg = _make_scalar_pass(0, _deg_val, False)
_sc_escore = _make_scalar_pass(3, _escore_val, True)
_sc_agg = _make_scalar_pass(1, _agg_val, False)
_sc_dstadd = _make_dstadd(False)
_sc_dstadd_w = _make_dstadd(True)
_sc_max = _make_maxpass()


def _pad_cols(w):
    return jnp.pad(w, ((0, 0), (0, _D - w.shape[1])))


def kernel(x, edge_index, batch, W1, b1, Wc, bc, Wq, bq, Wa, ba, leW1, leb1,
           leW2, leb2, leW3, leb3, Wl1, bl1, Wl2, bl2):
    row, col = edge_index[0], edge_index[1]
    xp = jnp.pad(x, ((0, _NPAD - _N), (0, 0)))
    batchf = jnp.pad(batch.astype(jnp.float32), (0, _NPAD - _N),
                     constant_values=127.0)
    idxf = jnp.arange(_NPAD, dtype=jnp.float32)
    zeros_n = jnp.zeros((_NPAD,), jnp.float32)

    deg = _tc_sum32(_sc_deg(row, col, zeros_n)) + 1.0
    dinv = lax.rsqrt(deg)

    g1 = _tc_mm(xp, W1)
    S1 = _sc_dstadd(dinv[:, None] * g1, row, col).reshape(_NPAD, _D)
    h = jax.nn.relu(dinv[:, None] * S1 + (dinv * dinv)[:, None] * g1 + b1)

    g2 = _tc_mm(h, Wc)
    S2 = _sc_dstadd(dinv[:, None] * g2, row, col).reshape(_NPAD, _D)
    x_pool = dinv[:, None] * S2 + (dinv * dinv)[:, None] * g2 + bc

    pj = _tc_mm(x_pool, _pad_cols(Wa[_H:, :1]))[:, 0]
    xq_flat, pmax_e = _sc_max(x_pool, row, col, pj)
    X_q = jnp.maximum(xq_flat.reshape(_NPAD, _D), x_pool)

    qn = _tc_mm(_tc_mm(X_q, Wq) + bq, _pad_cols(Wa[:_H, :1]))[:, 0]
    t1 = qn + ba[0]
    sm = t1 + jnp.maximum(pmax_e, pj)
    m = jnp.where(sm > 0, sm, 0.2 * sm)

    zpart, e_x = _sc_escore(row, col, zeros_n, t1, pj, m)
    zs = _tc_sum32(zpart)
    s_self = jnp.where(t1 + pj > 0, t1 + pj, 0.2 * (t1 + pj))
    e_self = jnp.exp(s_self - m)
    zinv = 1.0 / (zs + e_self + 1e-16)

    Xn = _sc_dstadd_w(h, row, col, e_x).reshape(_NPAD, _D)
    x_new = (Xn + e_self[:, None] * h) * zinv[:, None]

    le = _tc_mm(x_new, _pad_cols(jnp.concatenate([leW1, leW2, leW3], axis=1)))
    a = le[:, 0] + leb1[0]
    bb = le[:, 1] + leb2[0]
    cc = le[:, 2] + leb3[0]
    agg = _tc_sum32(_sc_agg(row, col, zeros_n, a)) + a
    fitness = jax.nn.sigmoid(cc + agg - deg * bb)

    key = batchf * 2.0 + (1.0 - fitness)
    keep = _tc_rank(key, batchf, idxf)

    x_out = x_new * fitness[:, None]
    xs = _tc_readout(x_out, keep, batchf)
    return _head(xs, Wl1, bl1, Wl2, bl2)


# speedup vs baseline: 3.0416x; 3.0416x over previous
import functools
import jax
import jax.numpy as jnp
from jax import lax
from jax.experimental import pallas as pl
from jax.experimental.pallas import tpu as pltpu
from jax.experimental.pallas import tpu_sc as plsc

_N = 10000
_NPAD = 10240
_E = 320000
_D = 128
_G = 64
_H = 128
_RATIO = 0.8
_NW = 32
_EW = _E // _NW
_TPW = _NPAD // _NW

_mesh = functools.partial(plsc.VectorSubcoreMesh,
                          core_axis_name="c", subcore_axis_name="s")
_cparams = functools.partial(pltpu.CompilerParams, needs_layout_passes=False)


def _wid():
    return lax.axis_index("c") * 16 + lax.axis_index("s")


def _splat_i32(x):
    return jnp.full((16,), x, jnp.int32)


def _make_scalar_pass(n_tables, val_fn, edge_out):
    CH = 2000
    NCH = _EW // CH

    def body(*refs):
        row_hbm, col_hbm, zeros_hbm = refs[0], refs[1], refs[2]
        tab_hbm = refs[3:3 + n_tables]
        i = 3 + n_tables
        out_hbm = refs[i]
        oute_hbm = refs[i + 1] if edge_out else None
        i += 2 if edge_out else 1
        rowbuf, colbuf, valbuf = refs[i], refs[i + 1], refs[i + 2]
        tabs = refs[i + 3:i + 3 + n_tables]
        acc = refs[i + 3 + n_tables]

        cid = lax.axis_index("c")
        sid = lax.axis_index("s")
        stripe = _NPAD // 16
        for t_hbm, t_v in zip(tab_hbm, tabs):
            pltpu.sync_copy(t_hbm, t_v)
        pltpu.sync_copy(zeros_hbm.at[pl.ds(sid * stripe, stripe)],
                        acc.at[pl.ds(sid * stripe, stripe)])
        plsc.subcore_barrier()

        base = _wid() * _EW

        def chunk(ci, c):
            pltpu.sync_copy(row_hbm.at[pl.ds(base + ci * CH, CH)], rowbuf)
            pltpu.sync_copy(col_hbm.at[pl.ds(base + ci * CH, CH)], colbuf)

            def grp(j, c2):
                r16 = rowbuf[pl.ds(j * 16, 16)]
                c16 = colbuf[pl.ds(j * 16, 16)]
                valbuf[pl.ds(j * 16, 16)] = val_fn(r16, c16, tabs)
                return c2
            lax.fori_loop(0, CH // 16, grp, 0)
            pltpu.sync_copy(valbuf, acc.at[colbuf], add=True)
            if edge_out:
                pltpu.sync_copy(valbuf,
                                oute_hbm.at[pl.ds(base + ci * CH, CH)])
            return c
        lax.fori_loop(0, NCH, chunk, 0)
        plsc.subcore_barrier()
        pltpu.sync_copy(acc.at[pl.ds(sid * stripe, stripe)],
                        out_hbm.at[cid, pl.ds(sid * stripe, stripe)])

    scratch = ([pltpu.VMEM((CH,), jnp.int32), pltpu.VMEM((CH,), jnp.int32),
                pltpu.VMEM((CH,), jnp.float32)]
               + [pltpu.VMEM((_NPAD,), jnp.float32) for _ in range(n_tables)]
               + [pltpu.VMEM_SHARED((_NPAD,), jnp.float32)])
    out_type = jax.ShapeDtypeStruct((2, _NPAD), jnp.float32)
    if edge_out:
        out_type = (out_type, jax.ShapeDtypeStruct((_E,), jnp.float32))
    return pl.kernel(
        body,
        out_type=out_type,
        mesh=_mesh(),
        scratch_types=scratch,
        compiler_params=_cparams(),
    )


def _deg_val(r16, c16, tabs):
    del r16, c16, tabs
    return jnp.ones((16,), jnp.float32)


def _escore_val(r16, c16, tabs):
    t1, pj, m = tabs
    s = plsc.load_gather(t1, [c16]) + plsc.load_gather(pj, [r16])
    s = jnp.where(s > 0, s, jnp.float32(0.2) * s)
    return jnp.exp(s - plsc.load_gather(m, [c16]))


def _agg_val(r16, c16, tabs):
    del c16
    return plsc.load_gather(tabs[0], [r16])


_CHS = 10000
_NR = _E // _CHS
_GB = 128


def _make_dstadd(weighted):
    def body(*refs):
        if weighted:
            z_hbm, row_hbm, col_hbm, w_hbm, out_hbm = refs[:5]
            rowb, colb, wb, crow, clc, cw, rows, acc, sem = refs[5:]
        else:
            z_hbm, row_hbm, col_hbm, out_hbm = refs[:4]
            rowb, colb, crow, clc, rows, acc, sem = refs[4:]
            wb = cw = None

        zero16f = jnp.zeros((16,), jnp.float32)
        zero16i = jnp.zeros((16,), jnp.int32)

        def iacc(i, c):
            acc[pl.ds(i * 16, 16)] = zero16f
            return c
        lax.fori_loop(0, _TPW * _D // 16, iacc, 0)

        def icrow(i, c):
            crow[pl.ds(i * 16, 16)] = zero16i
            return c
        lax.fori_loop(0, _CHS // 16, icrow, 0)

        lo = _wid() * _TPW
        hi = lo + _TPW

        def rnd(ri, c0):
            pltpu.sync_copy(row_hbm.at[pl.ds(ri * _CHS, _CHS)], rowb)
            pltpu.sync_copy(col_hbm.at[pl.ds(ri * _CHS, _CHS)], colb)
            if weighted:
                pltpu.sync_copy(w_hbm.at[pl.ds(ri * _CHS, _CHS)], wb)

            def grp(j, cnt):
                c16 = colb[pl.ds(j * 16, 16)]
                r16 = rowb[pl.ds(j * 16, 16)]
                msk = (c16 >= lo) & (c16 < hi)
                pos = (plsc.cumsum(msk.astype(jnp.int32))
                       + _splat_i32(cnt - 1))
                plsc.store_scatter(crow, [pos], r16, mask=msk)
                plsc.store_scatter(clc, [pos], c16 - lo, mask=msk)
                if weighted:
                    plsc.store_scatter(cw, [pos], wb[pl.ds(j * 16, 16)],
                                       mask=msk)
                return cnt + jnp.max(plsc.all_reduce_population_count(msk))
            cnt = lax.fori_loop(0, _CHS // 16, grp, 0)

            def batch(b, c2):
                s = b * _GB
                pltpu.async_copy(z_hbm.at[crow.at[pl.ds(s, _GB)]],
                                 rows, sem).wait()
                size = jnp.minimum(_GB, cnt - s)

                def edge(e, c3):
                    se16 = _splat_i32(s + e)
                    lc = plsc.load_gather(clc, [se16])
                    e16 = _splat_i32(e)
                    wv = plsc.load_gather(cw, [se16]) if weighted else None
                    for f in range(_D // 16):
                        fidx = f * 16 + lax.iota(jnp.int32, 16)
                        cur = plsc.load_gather(acc, [lc * _D + fidx])
                        rv = plsc.load_gather(rows, [e16, fidx])
                        if weighted:
                            rv = rv * wv
                        plsc.store_scatter(acc, [lc * _D + fidx], cur + rv)
                    return c3
                lax.fori_loop(0, size, edge, 0)
                return c2
            lax.fori_loop(0, (cnt + _GB - 1) // _GB, batch, 0)
            return c0
        lax.fori_loop(0, _NR, rnd, 0)
        pltpu.sync_copy(acc, out_hbm.at[pl.ds(lo * _D, _TPW * _D)])

    scratch = [pltpu.VMEM((_CHS,), jnp.int32), pltpu.VMEM((_CHS,), jnp.int32)]
    if weighted:
        scratch.append(pltpu.VMEM((_CHS,), jnp.float32))
    scratch += [pltpu.VMEM((_CHS,), jnp.int32), pltpu.VMEM((_CHS,), jnp.int32)]
    if weighted:
        scratch.append(pltpu.VMEM((_CHS,), jnp.float32))
    scratch += [pltpu.VMEM((_GB, _D), jnp.float32),
                pltpu.VMEM((_TPW * _D,), jnp.float32),
                pltpu.SemaphoreType.DMA]
    return pl.kernel(
        body,
        out_type=jax.ShapeDtypeStruct((_NPAD * _D,), jnp.float32),
        mesh=_mesh(),
        scratch_types=scratch,
        compiler_params=_cparams(),
    )


def _make_maxpass():
    def body(xp_hbm, row_hbm, col_hbm, pj_hbm, outx_hbm, outp_hbm,
             rowb, colb, crow, clc, rows, pjtab, accx, accp, sem):
        pltpu.sync_copy(pj_hbm, pjtab)
        neg = jnp.full((16,), -jnp.inf, jnp.float32)
        zero16i = jnp.zeros((16,), jnp.int32)

        def iacc(i, c):
            accx[pl.ds(i * 16, 16)] = neg
            return c
        lax.fori_loop(0, _TPW * _D // 16, iacc, 0)

        def iaccp(i, c):
            accp[pl.ds(i * 16, 16)] = neg
            return c
        lax.fori_loop(0, _TPW // 16, iaccp, 0)

        def icrow(i, c):
            crow[pl.ds(i * 16, 16)] = zero16i
            return c
        lax.fori_loop(0, _CHS // 16, icrow, 0)

        lo = _wid() * _TPW
        hi = lo + _TPW

        def rnd(ri, c):
            pltpu.sync_copy(row_hbm.at[pl.ds(ri * _CHS, _CHS)], rowb)
            pltpu.sync_copy(col_hbm.at[pl.ds(ri * _CHS, _CHS)], colb)

            def grp(j, cnt):
                c16 = colb[pl.ds(j * 16, 16)]
                r16 = rowb[pl.ds(j * 16, 16)]
                msk = (c16 >= lo) & (c16 < hi)
                pos = (plsc.cumsum(msk.astype(jnp.int32))
                       + _splat_i32(cnt - 1))
                plsc.store_scatter(crow, [pos], r16, mask=msk)
                plsc.store_scatter(clc, [pos], c16 - lo, mask=msk)
                return cnt + jnp.max(plsc.all_reduce_population_count(msk))
            cnt = lax.fori_loop(0, _CHS // 16, grp, 0)

            def batch(b, c2):
                s = b * _GB
                pltpu.async_copy(xp_hbm.at[crow.at[pl.ds(s, _GB)]],
                                 rows, sem).wait()
                size = jnp.minimum(_GB, cnt - s)

                def edge(e, c3):
                    se16 = _splat_i32(s + e)
                    lc = plsc.load_gather(clc, [se16])
                    crow16 = plsc.load_gather(crow, [se16])
                    e16 = _splat_i32(e)
                    for f in range(_D // 16):
                        fidx = f * 16 + lax.iota(jnp.int32, 16)
                        cur = plsc.load_gather(accx, [lc * _D + fidx])
                        rv = plsc.load_gather(rows, [e16, fidx])
                        plsc.store_scatter(accx, [lc * _D + fidx],
                                           jnp.maximum(cur, rv))
                    pv = plsc.load_gather(pjtab, [crow16])
                    cp = plsc.load_gather(accp, [lc])
                    plsc.store_scatter(accp, [lc], jnp.maximum(cp, pv))
                    return c3
                lax.fori_loop(0, size, edge, 0)
                return c2
            lax.fori_loop(0, (cnt + _GB - 1) // _GB, batch, 0)
            return c
        lax.fori_loop(0, _NR, rnd, 0)

        pltpu.sync_copy(accx, outx_hbm.at[pl.ds(lo * _D, _TPW * _D)])
        pltpu.sync_copy(accp, outp_hbm.at[pl.ds(lo, _TPW)])

    return pl.kernel(
        body,
        out_type=(jax.ShapeDtypeStruct((_NPAD * _D,), jnp.float32),
                  jax.ShapeDtypeStruct((_NPAD,), jnp.float32)),
        mesh=_mesh(),
        scratch_types=[
            pltpu.VMEM((_CHS,), jnp.int32), pltpu.VMEM((_CHS,), jnp.int32),
            pltpu.VMEM((_CHS,), jnp.int32), pltpu.VMEM((_CHS,), jnp.int32),
            pltpu.VMEM((_GB, _D), jnp.float32),
            pltpu.VMEM((_NPAD,), jnp.float32),
            pltpu.VMEM((_TPW * _D,), jnp.float32),
            pltpu.VMEM((_TPW,), jnp.float32),
            pltpu.SemaphoreType.DMA,
        ],
        compiler_params=_cparams(),
    )


_MMB = 2048


def _mm_kernel(x_ref, w_ref, o_ref):
    o_ref[...] = jnp.dot(x_ref[...], w_ref[...],
                         preferred_element_type=jnp.float32)


def _tc_mm(x, w):
    M = x.shape[0]
    blk = _MMB if M % _MMB == 0 else M
    return pl.pallas_call(
        _mm_kernel,
        grid=(M // blk,),
        in_specs=[pl.BlockSpec((blk, x.shape[1]), lambda i: (i, 0)),
                  pl.BlockSpec(w.shape, lambda i: (0, 0))],
        out_specs=pl.BlockSpec((blk, w.shape[1]), lambda i: (i, 0)),
        out_shape=jax.ShapeDtypeStruct((M, w.shape[1]), jnp.float32),
    )(x, w)


def _sum32_kernel(p_ref, o_ref):
    o_ref[...] = jnp.sum(p_ref[...], axis=0, keepdims=True)


def _tc_sum32(p):
    return pl.pallas_call(
        _sum32_kernel,
        out_shape=jax.ShapeDtypeStruct((1, _NPAD), jnp.float32),
    )(p)[0]


_RBI = 1024
_RBJ = 512


def _rank_kernel(kc_ref, bc_ref, ic_ref, kr_ref, br_ref, ir_ref, o_ref):
    ki, bi, ii = kc_ref[...], bc_ref[...], ic_ref[...]
    kl = jnp.zeros((_RBI, 1), jnp.float32)
    sb = jnp.zeros((_RBI, 1), jnp.float32)
    cb = jnp.zeros((_RBI, 1), jnp.float32)

    def jstep(j, carry):
        kl, sb, cb = carry
        kj = kr_ref[:, pl.ds(j * _RBJ, _RBJ)]
        bj = br_ref[:, pl.ds(j * _RBJ, _RBJ)]
        ij = ir_ref[:, pl.ds(j * _RBJ, _RBJ)]
        less = (kj < ki) | ((kj == ki) & (ij < ii))
        kl = kl + jnp.sum(less.astype(jnp.float32), axis=1, keepdims=True)
        sb = sb + jnp.sum((bj < bi).astype(jnp.float32), axis=1,
                          keepdims=True)
        cb = cb + jnp.sum((bj == bi).astype(jnp.float32), axis=1,
                          keepdims=True)
        return kl, sb, cb
    kl, sb, cb = lax.fori_loop(0, _NPAD // _RBJ, jstep, (kl, sb, cb))
    kper = jnp.ceil(jnp.float32(_RATIO) * cb)
    o_ref[...] = ((kl - sb) < kper).astype(jnp.float32)


def _tc_rank(key, batchf, idxf):
    kc = key[:, None]
    bc = batchf[:, None]
    ic = idxf[:, None]
    kr = key[None, :]
    br = batchf[None, :]
    ir = idxf[None, :]
    return pl.pallas_call(
        _rank_kernel,
        grid=(_NPAD // _RBI,),
        in_specs=[pl.BlockSpec((_RBI, 1), lambda i: (i, 0)),
                  pl.BlockSpec((_RBI, 1), lambda i: (i, 0)),
                  pl.BlockSpec((_RBI, 1), lambda i: (i, 0)),
                  pl.BlockSpec((1, _NPAD), lambda i: (0, 0)),
                  pl.BlockSpec((1, _NPAD), lambda i: (0, 0)),
                  pl.BlockSpec((1, _NPAD), lambda i: (0, 0))],
        out_specs=pl.BlockSpec((_RBI, 1), lambda i: (i, 0)),
        out_shape=jax.ShapeDtypeStruct((_NPAD, 1), jnp.float32),
    )(kc, bc, ic, kr, br, ir)[:, 0]


def _readout_kernel(x_ref, keep_ref, b_ref, o_ref):
    g0 = pl.program_id(0) * 8
    x = x_ref[...]
    keepm = keep_ref[...] > 0.5
    b = b_ref[...]
    rows = []
    for k in range(8):
        gval = (g0 + k).astype(jnp.float32)
        mask = (b == gval) & keepm
        kcnt = jnp.sum(mask.astype(jnp.float32))
        xm = jnp.sum(jnp.where(mask, x, 0.0), axis=0)
        xmx = jnp.max(jnp.where(mask, x, -jnp.inf), axis=0)
        rows.append(jnp.concatenate([xm / jnp.maximum(kcnt, 1.0), xmx],
                                    axis=-1))
    o_ref[...] = jnp.stack(rows)


def _tc_readout(x_out, keep, batchf):
    return pl.pallas_call(
        _readout_kernel,
        grid=(_G // 8,),
        in_specs=[pl.BlockSpec((_NPAD, _D), lambda g: (0, 0)),
                  pl.BlockSpec((_NPAD, 1), lambda g: (0, 0)),
                  pl.BlockSpec((_NPAD, 1), lambda g: (0, 0))],
        out_specs=pl.BlockSpec((8, 2 * _D), lambda g: (g, 0)),
        out_shape=jax.ShapeDtypeStruct((_G, 2 * _D), jnp.float32),
    )(x_out, keep[:, None], batchf[:, None])


def _head_kernel(xs_ref, wl1_ref, bl1_ref, wl2_ref, bl2_ref, out_ref):
    xs = xs_ref[...]
    z = jnp.maximum(jnp.dot(xs, wl1_ref[...],
                            preferred_element_type=jnp.float32)
                    + bl1_ref[...], 0.0)
    z = jnp.dot(z, wl2_ref[...], preferred_element_type=jnp.float32) \
        + bl2_ref[...]
    m = jnp.max(z, axis=-1, keepdims=True)
    e = jnp.exp(z - m)
    lse = jnp.log(jnp.sum(e, axis=-1, keepdims=True)) + m
    out_ref[...] = z - lse


def _head(xs, Wl1, bl1, Wl2, bl2):
    return pl.pallas_call(
        _head_kernel,
        out_shape=jax.ShapeDtypeStruct((_G, Wl2.shape[1]), jnp.float32),
    )(xs, Wl1, bl1[None, :], Wl2, bl2[None, :])


_sc_deg = _make_scalar_pass(0, _deg_val, False)
_sc_escore = _make_scalar_pass(3, _escore_val, True)
_sc_agg = _make_scalar_pass(1, _agg_val, False)
_sc_dstadd = _make_dstadd(False)
_sc_dstadd_w = _make_dstadd(True)
_sc_max = _make_maxpass()


def _pad_cols(w):
    return jnp.pad(w, ((0, 0), (0, _D - w.shape[1])))


def kernel(x, edge_index, batch, W1, b1, Wc, bc, Wq, bq, Wa, ba, leW1, leb1,
           leW2, leb2, leW3, leb3, Wl1, bl1, Wl2, bl2):
    row, col = edge_index[0], edge_index[1]
    xp = jnp.pad(x, ((0, _NPAD - _N), (0, 0)))
    batchf = jnp.pad(batch.astype(jnp.float32), (0, _NPAD - _N),
                     constant_values=127.0)
    idxf = jnp.arange(_NPAD, dtype=jnp.float32)
    zeros_n = jnp.zeros((_NPAD,), jnp.float32)

    deg = _tc_sum32(_sc_deg(row, col, zeros_n)) + 1.0
    dinv = lax.rsqrt(deg)

    g1 = _tc_mm(xp, W1)
    S1 = _sc_dstadd(dinv[:, None] * g1, row, col).reshape(_NPAD, _D)
    h = jax.nn.relu(dinv[:, None] * S1 + (dinv * dinv)[:, None] * g1 + b1)

    g2 = _tc_mm(h, Wc)
    S2 = _sc_dstadd(dinv[:, None] * g2, row, col).reshape(_NPAD, _D)
    x_pool = dinv[:, None] * S2 + (dinv * dinv)[:, None] * g2 + bc

    pj = _tc_mm(x_pool, _pad_cols(Wa[_H:, :1]))[:, 0]
    xq_flat, pmax_e = _sc_max(x_pool, row, col, pj)
    X_q = jnp.maximum(xq_flat.reshape(_NPAD, _D), x_pool)

    qn = _tc_mm(_tc_mm(X_q, Wq) + bq, _pad_cols(Wa[:_H, :1]))[:, 0]
    t1 = qn + ba[0]
    sm = t1 + jnp.maximum(pmax_e, pj)
    m = jnp.where(sm > 0, sm, 0.2 * sm)

    zpart, e_x = _sc_escore(row, col, zeros_n, t1, pj, m)
    zs = _tc_sum32(zpart)
    s_self = jnp.where(t1 + pj > 0, t1 + pj, 0.2 * (t1 + pj))
    e_self = jnp.exp(s_self - m)
    zinv = 1.0 / (zs + e_self + 1e-16)

    Xn = _sc_dstadd_w(h, row, col, e_x).reshape(_NPAD, _D)
    x_new = (Xn + e_self[:, None] * h) * zinv[:, None]

    le = _tc_mm(x_new, _pad_cols(jnp.concatenate([leW1, leW2, leW3], axis=1)))
    a = le[:, 0] + leb1[0]
    bb = le[:, 1] + leb2[0]
    cc = le[:, 2] + leb3[0]
    agg = _tc_sum32(_sc_agg(row, col, zeros_n, a)) + a
    fitness = jax.nn.sigmoid(cc + agg - deg * bb)

    key = batchf * 2.0 + (1.0 - fitness)
    keep = _tc_rank(key, batchf, idxf)

    x_out = x_new * fitness[:, None]
    xs = _tc_readout(x_out, keep, batchf)
    return _head(xs, Wl1, bl1, Wl2, bl2)
